# Initial kernel scaffold; baseline (speedup 1.0000x reference)
import functools

import jax
import jax.numpy as jnp
from jax import lax
from jax.experimental import pallas as pl
from jax.experimental.pallas import tpu as pltpu
from jax.experimental.pallas import tpu_sc as plsc

DIM = 32
NW = 32
CHUNK = 1024


def _emb_call(flat_idx, embed_weight):
    B = flat_idx.shape[0]
    bpw = B // NW
    nch = bpw // CHUNK
    assert bpw % CHUNK == 0 and B % NW == 0

    mesh = plsc.VectorSubcoreMesh(core_axis_name="c", subcore_axis_name="s")

    @functools.partial(
        pl.kernel,
        mesh=mesh,
        out_type=jax.ShapeDtypeStruct((B, DIM), jnp.float32),
        scratch_types=[
            pltpu.VMEM((bpw,), jnp.int32),
            pltpu.VMEM((CHUNK, DIM), jnp.float32),
            pltpu.SemaphoreType.DMA,
        ],
    )
    def emb(idx_hbm, table_hbm, out_hbm, idx_v, rows_v, sem):
        wid = lax.axis_index("s") * 2 + lax.axis_index("c")
        base = wid * bpw
        pltpu.sync_copy(idx_hbm.at[pl.ds(base, bpw)], idx_v)

        def body(c, carry):
            off = c * CHUNK
            pltpu.async_copy(
                table_hbm.at[idx_v.at[pl.ds(off, CHUNK)]],
                rows_v,
                sem,
            ).wait()
            pltpu.sync_copy(rows_v, out_hbm.at[pl.ds(base + off, CHUNK)])
            return carry

        lax.fori_loop(0, nch, body, 0)

    return emb(flat_idx, embed_weight)


def kernel(word_index, embed_weight):
    b, f = word_index.shape
    flat = jnp.reshape(word_index.astype(jnp.int32), (b * f,))
    out = _emb_call(flat, embed_weight)
    return jnp.reshape(out, (b, f, DIM))


# speedup vs baseline: 1.5603x; 1.5603x over previous
import functools

import jax
import jax.numpy as jnp
from jax import lax
from jax.experimental import pallas as pl
from jax.experimental.pallas import tpu as pltpu
from jax.experimental.pallas import tpu_sc as plsc

DIM = 32
NW = 32
CHUNK = 1024


def _emb_call(flat_idx, embed_weight):
    B = flat_idx.shape[0]
    bpw = B // NW
    nch = bpw // CHUNK
    assert bpw % CHUNK == 0 and B % NW == 0

    mesh = plsc.VectorSubcoreMesh(core_axis_name="c", subcore_axis_name="s")

    @functools.partial(
        pl.kernel,
        mesh=mesh,
        out_type=jax.ShapeDtypeStruct((B, DIM), jnp.float32),
        compiler_params=pltpu.CompilerParams(use_tc_tiling_on_sc=False),
        scratch_types=[
            pltpu.VMEM((bpw,), jnp.int32),
            pltpu.VMEM((CHUNK, DIM), jnp.float32),
            pltpu.SemaphoreType.DMA,
        ],
    )
    def emb(idx_hbm, table_hbm, out_hbm, idx_v, rows_v, sem):
        wid = lax.axis_index("s") * 2 + lax.axis_index("c")
        base = wid * bpw
        pltpu.sync_copy(idx_hbm.at[pl.ds(base, bpw)], idx_v)

        def body(c, carry):
            off = c * CHUNK
            pltpu.async_copy(
                table_hbm.at[idx_v.at[pl.ds(off, CHUNK)]],
                rows_v,
                sem,
            ).wait()
            pltpu.sync_copy(rows_v, out_hbm.at[pl.ds(base + off, CHUNK)])
            return carry

        lax.fori_loop(0, nch, body, 0)

    return emb(flat_idx, embed_weight)


def kernel(word_index, embed_weight):
    b, f = word_index.shape
    flat = jnp.reshape(word_index.astype(jnp.int32), (b * f,))
    out = _emb_call(flat, embed_weight)
    return jnp.reshape(out, (b, f, DIM))


# speedup vs baseline: 1.5707x; 1.0066x over previous
import functools

import jax
import jax.numpy as jnp
from jax import lax
from jax.experimental import pallas as pl
from jax.experimental.pallas import tpu as pltpu
from jax.experimental.pallas import tpu_sc as plsc

DIM = 32
NW = 32
NBUF = 2
CHUNK = 1664


def _emb_call(flat_idx, embed_weight):
    B = flat_idx.shape[0]
    bpw = B // NW
    nst = bpw // (CHUNK * NBUF)
    assert bpw % (CHUNK * NBUF) == 0 and B % NW == 0

    mesh = plsc.VectorSubcoreMesh(core_axis_name="c", subcore_axis_name="s")

    @functools.partial(
        pl.kernel,
        mesh=mesh,
        out_type=jax.ShapeDtypeStruct((B, DIM), jnp.float32),
        compiler_params=pltpu.CompilerParams(use_tc_tiling_on_sc=False),
        scratch_types=[
            pltpu.VMEM((bpw,), jnp.int32),
            pltpu.VMEM((NBUF, CHUNK, DIM), jnp.float32),
            [pltpu.SemaphoreType.DMA] * NBUF,
            [pltpu.SemaphoreType.DMA] * NBUF,
        ],
    )
    def emb(idx_hbm, table_hbm, out_hbm, idx_v, rows_v, gsems, wsems):
        wid = lax.axis_index("s") * 2 + lax.axis_index("c")
        base = wid * bpw
        pltpu.sync_copy(idx_hbm.at[pl.ds(base, bpw)], idx_v)

        def gather(step, b, start):
            off = (step * NBUF + b) * CHUNK
            desc = pltpu.make_async_copy(
                table_hbm.at[idx_v.at[pl.ds(off, CHUNK)]],
                rows_v.at[b],
                gsems[b],
            )
            if start:
                desc.start()
            else:
                desc.wait()

        def write(step, b, start):
            off = (step * NBUF + b) * CHUNK
            desc = pltpu.make_async_copy(
                rows_v.at[b],
                out_hbm.at[pl.ds(base + off, CHUNK)],
                wsems[b],
            )
            if start:
                desc.start()
            else:
                desc.wait()

        for b in range(NBUF):
            gather(0, b, start=True)

        def body(step, carry):
            for b in range(NBUF):
                gather(step, b, start=False)
                write(step, b, start=True)

            @pl.when(step + 1 < nst)
            def _():
                for b in range(NBUF):
                    write(step, b, start=False)
                    gather(step + 1, b, start=True)

            return carry

        lax.fori_loop(0, nst, body, 0)

        for b in range(NBUF):
            write(nst - 1, b, start=False)

    return emb(flat_idx, embed_weight)


def kernel(word_index, embed_weight):
    b, f = word_index.shape
    flat = jnp.reshape(word_index.astype(jnp.int32), (b * f,))
    out = _emb_call(flat, embed_weight)
    return jnp.reshape(out, (b, f, DIM))
